# Initial kernel scaffold; baseline (speedup 1.0000x reference)
import jax
import jax.numpy as jnp
from jax.experimental import pallas as pl


def kernel(clear_feature, rain_feature, W1, b1, W2, b2):
    raise NotImplementedError("write your pallas kernel here")


# speedup vs baseline: 4.0724x; 4.0724x over previous
import functools

import jax
import jax.numpy as jnp
from jax import lax
from jax.experimental import pallas as pl
from jax.experimental.pallas import tpu as pltpu
from jax.experimental.pallas import tpu_sc as plsc

_N = 4096
_M = 8192
_D = 512

_BI = 256
_BJ = 1024
_JBLKS = _M // _BJ


def _argmin_body(x2_ref, y2_ref, x_ref, y_ref, idx_ref, bestd_ref, besti_ref):
    j = pl.program_id(1)

    @pl.when(j == 0)
    def _init():
        bestd_ref[...] = jnp.full((_BI, 1), jnp.inf, jnp.float32)
        besti_ref[...] = jnp.zeros((_BI, 1), jnp.int32)

    mm = jax.lax.dot_general(
        x_ref[...], y_ref[...], (((1,), (1,)), ((), ())),
        preferred_element_type=jnp.float32)
    sq = (x2_ref[...] + y2_ref[...]) - 2.0 * mm
    dist = jnp.sqrt(jnp.maximum(sq, 0.0))

    m = jnp.min(dist, axis=1, keepdims=True)
    cols = lax.broadcasted_iota(jnp.int32, (_BI, _BJ), 1) + j * _BJ
    cand = jnp.min(jnp.where(dist == m, cols, _M), axis=1, keepdims=True)

    upd = m < bestd_ref[...]
    besti_ref[...] = jnp.where(upd, cand, besti_ref[...])
    bestd_ref[...] = jnp.where(upd, m, bestd_ref[...])

    @pl.when(j == _JBLKS - 1)
    def _fin():
        idx_ref[...] = besti_ref[...]


def _nearest_idx(x, y, x2, y2):
    return pl.pallas_call(
        _argmin_body,
        grid=(_N // _BI, _JBLKS),
        in_specs=[
            pl.BlockSpec((_BI, 1), lambda i, j: (i, 0)),
            pl.BlockSpec((1, _BJ), lambda i, j: (0, j)),
            pl.BlockSpec((_BI, _D), lambda i, j: (i, 0)),
            pl.BlockSpec((_BJ, _D), lambda i, j: (j, 0)),
        ],
        out_specs=pl.BlockSpec((_BI, 1), lambda i, j: (i, 0)),
        out_shape=jax.ShapeDtypeStruct((_N, 1), jnp.int32),
        scratch_shapes=[
            pltpu.VMEM((_BI, 1), jnp.float32),
            pltpu.VMEM((_BI, 1), jnp.int32),
        ],
        compiler_params=pltpu.CompilerParams(
            dimension_semantics=("parallel", "arbitrary")),
    )(x2, y2, x, y)


def _make_sc_gather():
    info = plsc.get_sparse_core_info()
    nc, ns = info.num_cores, info.num_subcores
    nw = nc * ns
    b_per_w = _N // nw
    mesh = plsc.VectorSubcoreMesh(core_axis_name="c", subcore_axis_name="s")

    @functools.partial(
        pl.kernel, mesh=mesh,
        out_type=jax.ShapeDtypeStruct((_N, _D), jnp.float32),
        scratch_types=[
            pltpu.VMEM((b_per_w,), jnp.int32),
            pltpu.VMEM((b_per_w, _D), jnp.float32),
            pltpu.SemaphoreType.DMA,
        ],
    )
    def _gather(table_hbm, idx_hbm, out_hbm, idx_v, rows_v, sem):
        wid = lax.axis_index("s") * nc + lax.axis_index("c")
        base = wid * b_per_w
        pltpu.sync_copy(idx_hbm.at[pl.ds(base, b_per_w)], idx_v)
        pltpu.async_copy(table_hbm.at[idx_v], rows_v, sem).wait()
        pltpu.sync_copy(rows_v, out_hbm.at[pl.ds(base, b_per_w)])

    return _gather


_BF = 512


def _fuse_body(x_ref, a_ref, w1a_ref, w1b_ref, b1_ref, w2_ref, b2_ref, o_ref):
    x = x_ref[...]
    a = a_ref[...]
    h = (jnp.dot(x, w1a_ref[...], preferred_element_type=jnp.float32)
         + jnp.dot(a, w1b_ref[...], preferred_element_type=jnp.float32)
         + b1_ref[...])
    h = jnp.maximum(h, 0.0)
    z = jnp.dot(h, w2_ref[...], preferred_element_type=jnp.float32) + b2_ref[...]
    w = jax.nn.sigmoid(z[:, 0:1])
    o_ref[...] = w * x + (1.0 - w) * a


def _fuse(x, aligned, w1a, w1b, b1, w2, b2):
    return pl.pallas_call(
        _fuse_body,
        grid=(_N // _BF,),
        in_specs=[
            pl.BlockSpec((_BF, _D), lambda i: (i, 0)),
            pl.BlockSpec((_BF, _D), lambda i: (i, 0)),
            pl.BlockSpec((_D, _D), lambda i: (0, 0)),
            pl.BlockSpec((_D, _D), lambda i: (0, 0)),
            pl.BlockSpec((1, _D), lambda i: (0, 0)),
            pl.BlockSpec((_D, 128), lambda i: (0, 0)),
            pl.BlockSpec((1, 128), lambda i: (0, 0)),
        ],
        out_specs=pl.BlockSpec((_BF, _D), lambda i: (i, 0)),
        out_shape=jax.ShapeDtypeStruct((_N, _D), jnp.float32),
        compiler_params=pltpu.CompilerParams(
            dimension_semantics=("parallel",)),
    )(x, aligned, w1a, w1b, b1, w2, b2)


def kernel(clear_feature, rain_feature, W1, b1, W2, b2):
    x, y = clear_feature, rain_feature
    x2 = jnp.sum(x * x, axis=-1, keepdims=True)
    y2 = jnp.sum(y * y, axis=-1, keepdims=True).T

    idx = _nearest_idx(x, y, x2, y2)
    aligned = _make_sc_gather()(y, idx.reshape(_N))

    w1a = W1[:_D]
    w1b = W1[_D:]
    w2p = jnp.zeros((_D, 128), jnp.float32).at[:, 0].set(W2[:, 0])
    b2p = jnp.broadcast_to(b2.reshape(1, 1), (1, 128))
    return _fuse(x, aligned, w1a, w1b, b1.reshape(1, _D), w2p, b2p)
